# Initial kernel scaffold; baseline (speedup 1.0000x reference)
import jax
import jax.numpy as jnp
from jax.experimental import pallas as pl


def kernel(node_attrs, node_feats, edge_attrs, edge_feats, edge_index, W_up, W_mlp1, W_mlp2, W_mlp3, W_mlp4, W_lin, W_skip, W_dens, W_demb):
    raise NotImplementedError("write your pallas kernel here")


# speedup vs baseline: 1.0696x; 1.0696x over previous
import functools
import math

import jax
import jax.numpy as jnp
from jax import lax
from jax.experimental import pallas as pl
from jax.experimental.pallas import tpu as pltpu
from jax.experimental.pallas import tpu_sc as plsc


def _silu(x):
    return x * jax.nn.sigmoid(x)


def _up_kernel(nf_ref, w_ref, o_ref):
    o_ref[...] = jnp.dot(nf_ref[...], w_ref[...], precision="highest")


def _edge_kernel(ef_ref, ea_ref, w1_ref, w2_ref, w3_ref, w4_ref, wd_ref,
                 w_ref, dp_ref):
    ef = ef_ref[...]
    h = _silu(jnp.dot(ef, w1_ref[...], precision="highest"))
    h = _silu(jnp.dot(h, w2_ref[...], precision="highest"))
    h = _silu(jnp.dot(h, w3_ref[...], precision="highest"))
    tw = jnp.dot(h, w4_ref[...], precision="highest")
    w_ref[...] = tw * ea_ref[...]
    dp_ref[...] = jnp.tanh(jnp.dot(ef, wd_ref[...], precision="highest") ** 2)


def _tail_kernel(msgp_ref, densp_ref, attr_ref, fr_ref, wsin_ref, wcos_ref,
                 wlin_ref, ws_ref, o_ref):
    msg = msgp_ref[0] + msgp_ref[1]
    dens = densp_ref[0] + densp_ref[1]
    message = msg / (dens + 1.0)
    args = dens * fr_ref[...]
    message = (message
               + jnp.dot(jnp.sin(args), wsin_ref[...], precision="highest")
               + jnp.dot(jnp.cos(args), wcos_ref[...], precision="highest"))
    t = jnp.dot(message, wlin_ref[...], precision="highest")
    acc = jnp.dot(t, ws_ref[0], precision="highest") * attr_ref[:, 0:1]
    for v in range(1, ws_ref.shape[0]):
        acc = acc + (jnp.dot(t, ws_ref[v], precision="highest")
                     * attr_ref[:, v:v + 1])
    o_ref[...] = acc


def _make_sc_scatter(n_nodes, d_feat, n_edges, nc, ns):
    nw = nc * ns
    ept = n_edges // nw
    K = 40
    n_chunks = ept // K
    share = (n_nodes // ns) // 8 * 8
    extra = n_nodes - share * ns
    ZR = 16
    nz = share // ZR

    mesh = plsc.VectorSubcoreMesh(core_axis_name="c", subcore_axis_name="s")

    @functools.partial(
        pl.kernel,
        mesh=mesh,
        out_type=[
            jax.ShapeDtypeStruct((nc, n_nodes, d_feat), jnp.float32),
            jax.ShapeDtypeStruct((nc * n_nodes,), jnp.float32),
        ],
        scratch_types=[
            pltpu.VMEM((K, d_feat), jnp.float32),
            pltpu.VMEM((K, d_feat), jnp.float32),
            pltpu.VMEM((K,), jnp.float32),
            pltpu.VMEM((K,), jnp.int32),
            pltpu.VMEM((K,), jnp.int32),
            pltpu.VMEM((ZR, d_feat), jnp.float32),
            pltpu.VMEM((share,), jnp.float32),
            pltpu.VMEM_SHARED((n_nodes, d_feat), jnp.float32),
            pltpu.VMEM_SHARED((n_nodes,), jnp.float32),
            pltpu.SemaphoreType.DMA,
        ],
    )
    def sc_scatter(x_hbm, w_hbm, d_hbm, snd_hbm, rcv_hbm,
                   msg_out, dens_out,
                   xrows, wrows, drows, sidx, ridx, zbuf, zdbuf,
                   msg_acc, dens_acc, sem):
        cid = lax.axis_index("c")
        sid = lax.axis_index("s")
        wid = sid * nc + cid

        zero16 = jnp.zeros((16,), jnp.float32)

        def zb_body(r, _):
            for g in range(d_feat // 16):
                zbuf[r, pl.ds(g * 16, 16)] = zero16
            return 0
        lax.fori_loop(0, ZR, zb_body, 0)

        def zd_body(r, _):
            zdbuf[pl.ds(r * 16, 16)] = zero16
            return 0
        lax.fori_loop(0, share // 16, zd_body, 0)

        row0 = sid * share
        for t in range(nz):
            pltpu.sync_copy(zbuf, msg_acc.at[pl.ds(row0 + t * ZR, ZR), :])
        pltpu.sync_copy(zdbuf, dens_acc.at[pl.ds(row0, share)])

        @pl.when(sid == ns - 1)
        def _zero_tail():
            r = ns * share
            pltpu.sync_copy(zbuf.at[pl.ds(0, extra), :],
                            msg_acc.at[pl.ds(r, extra), :])
            pltpu.sync_copy(zdbuf.at[pl.ds(0, extra)],
                            dens_acc.at[pl.ds(r, extra)])
        plsc.subcore_barrier()

        base_edge = wid * ept

        def chunk(i, _):
            e0 = base_edge + i * K
            pltpu.sync_copy(snd_hbm.at[pl.ds(e0, K)], sidx)
            pltpu.sync_copy(rcv_hbm.at[pl.ds(e0, K)], ridx)
            pltpu.async_copy(x_hbm.at[sidx], xrows, sem).wait()
            pltpu.sync_copy(w_hbm.at[pl.ds(e0, K), :], wrows)
            pltpu.sync_copy(d_hbm.at[pl.ds(e0, K)], drows)

            def mulrow(r, _):
                for g in range(d_feat // 16):
                    c = g * 16
                    wrows[r, pl.ds(c, 16)] = (wrows[r, pl.ds(c, 16)]
                                              * xrows[r, pl.ds(c, 16)])
                return 0
            lax.fori_loop(0, K, mulrow, 0)

            pltpu.sync_copy(wrows, msg_acc.at[ridx], add=True)
            pltpu.sync_copy(drows, dens_acc.at[ridx], add=True)
            return 0
        lax.fori_loop(0, n_chunks, chunk, 0)

        plsc.subcore_barrier()
        pltpu.sync_copy(msg_acc.at[pl.ds(row0, share), :],
                        msg_out.at[cid, pl.ds(row0, share), :])
        pltpu.sync_copy(dens_acc.at[pl.ds(row0, share)], zdbuf)
        pltpu.sync_copy(zdbuf,
                        dens_out.at[pl.ds(cid * n_nodes + row0, share)])

        @pl.when(sid == ns - 1)
        def _pub_tail():
            r = ns * share
            pltpu.sync_copy(msg_acc.at[pl.ds(r, extra), :],
                            msg_out.at[cid, pl.ds(r, extra), :])
            pltpu.sync_copy(dens_acc.at[pl.ds(r, extra)],
                            zdbuf.at[pl.ds(0, extra)])
            pltpu.sync_copy(zdbuf.at[pl.ds(0, extra)],
                            dens_out.at[pl.ds(cid * n_nodes + r, extra)])

    return sc_scatter


def kernel(node_attrs, node_feats, edge_attrs, edge_feats, edge_index,
           W_up, W_mlp1, W_mlp2, W_mlp3, W_mlp4, W_lin, W_skip, W_dens,
           W_demb):
    n_nodes, d_feat = node_feats.shape
    n_edges, d_edge = edge_feats.shape
    d_attr = node_attrs.shape[1]

    f32 = jnp.float32
    w_up = W_up / math.sqrt(d_feat)
    w1 = W_mlp1 / math.sqrt(d_edge)
    w2 = W_mlp2 / math.sqrt(64.0)
    w3 = W_mlp3 / math.sqrt(64.0)
    w4 = W_mlp4 / math.sqrt(64.0)
    wd = W_dens / math.sqrt(d_edge)
    wlin = W_lin / math.sqrt(d_feat)
    wskip = jnp.transpose(W_skip, (1, 0, 2)) / math.sqrt(d_feat * d_attr)
    half = 16
    freqs = jnp.exp(-math.log(100.0)
                    * jnp.arange(half, dtype=f32) / half).reshape(1, half)
    wsin = W_demb[:half]
    wcos = W_demb[half:]

    BN = 1000
    x = pl.pallas_call(
        _up_kernel,
        grid=(n_nodes // BN,),
        in_specs=[
            pl.BlockSpec((BN, d_feat), lambda i: (i, 0)),
            pl.BlockSpec((d_feat, d_feat), lambda i: (0, 0)),
        ],
        out_specs=pl.BlockSpec((BN, d_feat), lambda i: (i, 0)),
        out_shape=jax.ShapeDtypeStruct((n_nodes, d_feat), f32),
    )(node_feats, w_up)

    EB = 2560
    w_edges, dens_edges = pl.pallas_call(
        _edge_kernel,
        grid=(n_edges // EB,),
        in_specs=[
            pl.BlockSpec((EB, d_edge), lambda i: (i, 0)),
            pl.BlockSpec((EB, 1), lambda i: (i, 0)),
            pl.BlockSpec((d_edge, 64), lambda i: (0, 0)),
            pl.BlockSpec((64, 64), lambda i: (0, 0)),
            pl.BlockSpec((64, 64), lambda i: (0, 0)),
            pl.BlockSpec((64, d_feat), lambda i: (0, 0)),
            pl.BlockSpec((d_edge, 1), lambda i: (0, 0)),
        ],
        out_specs=[
            pl.BlockSpec((EB, d_feat), lambda i: (i, 0)),
            pl.BlockSpec((EB, 1), lambda i: (i, 0)),
        ],
        out_shape=[
            jax.ShapeDtypeStruct((n_edges, d_feat), f32),
            jax.ShapeDtypeStruct((n_edges, 1), f32),
        ],
    )(edge_feats, edge_attrs, w1, w2, w3, w4, wd)

    info = plsc.get_sparse_core_info()
    nc, ns = info.num_cores, info.num_subcores
    sender = edge_index[0]
    receiver = edge_index[1]
    sc = _make_sc_scatter(n_nodes, d_feat, n_edges, nc, ns)
    msg_p, dens_p = sc(x, w_edges, dens_edges.reshape(n_edges),
                       sender, receiver)
    dens_p = dens_p.reshape(nc, n_nodes, 1)

    out = pl.pallas_call(
        _tail_kernel,
        grid=(n_nodes // BN,),
        in_specs=[
            pl.BlockSpec((nc, BN, d_feat), lambda i: (0, i, 0)),
            pl.BlockSpec((nc, BN, 1), lambda i: (0, i, 0)),
            pl.BlockSpec((BN, d_attr), lambda i: (i, 0)),
            pl.BlockSpec((1, half), lambda i: (0, 0)),
            pl.BlockSpec((half, d_feat), lambda i: (0, 0)),
            pl.BlockSpec((half, d_feat), lambda i: (0, 0)),
            pl.BlockSpec((d_feat, d_feat), lambda i: (0, 0)),
            pl.BlockSpec((d_attr, d_feat, d_feat), lambda i: (0, 0, 0)),
        ],
        out_specs=pl.BlockSpec((BN, d_feat), lambda i: (i, 0)),
        out_shape=jax.ShapeDtypeStruct((n_nodes, d_feat), f32),
    )(msg_p, dens_p, node_attrs, freqs, wsin, wcos, wlin, wskip)

    return out[:, :, None]


# speedup vs baseline: 1.3419x; 1.2546x over previous
import functools
import math

import jax
import jax.numpy as jnp
from jax import lax
from jax.experimental import pallas as pl
from jax.experimental.pallas import tpu as pltpu
from jax.experimental.pallas import tpu_sc as plsc


def _silu(x):
    return x * jax.nn.sigmoid(x)


def _up_kernel(nf_ref, w_ref, o_ref):
    o_ref[...] = jnp.dot(nf_ref[...], w_ref[...], precision="highest")


def _edge_kernel(ef_ref, ea_ref, w1_ref, w2_ref, w3_ref, w4_ref, wd_ref,
                 w_ref, dp_ref):
    ef = ef_ref[...]
    h = _silu(jnp.dot(ef, w1_ref[...], precision="highest"))
    h = _silu(jnp.dot(h, w2_ref[...], precision="highest"))
    h = _silu(jnp.dot(h, w3_ref[...], precision="highest"))
    tw = jnp.dot(h, w4_ref[...], precision="highest")
    w_ref[...] = tw * ea_ref[...]
    dp_ref[...] = jnp.tanh(jnp.dot(ef, wd_ref[...], precision="highest") ** 2)


def _tail_kernel(msgp_ref, densp_ref, attr_ref, fr_ref, wsin_ref, wcos_ref,
                 wlin_ref, ws_ref, o_ref):
    msg = msgp_ref[0] + msgp_ref[1]
    dens = densp_ref[0] + densp_ref[1]
    message = msg / (dens + 1.0)
    args = dens * fr_ref[...]
    message = (message
               + jnp.dot(jnp.sin(args), wsin_ref[...], precision="highest")
               + jnp.dot(jnp.cos(args), wcos_ref[...], precision="highest"))
    t = jnp.dot(message, wlin_ref[...], precision="highest")
    acc = jnp.dot(t, ws_ref[0], precision="highest") * attr_ref[:, 0:1]
    for v in range(1, ws_ref.shape[0]):
        acc = acc + (jnp.dot(t, ws_ref[v], precision="highest")
                     * attr_ref[:, v:v + 1])
    o_ref[...] = acc


def _make_sc_scatter(n_nodes, d_feat, n_edges, nc, ns):
    nw = nc * ns
    ept = n_edges // nw
    K = 40
    n_chunks = ept // K
    share = (n_nodes // ns) // 8 * 8
    extra = n_nodes - share * ns
    ZR = 16
    nz = share // ZR

    mesh = plsc.VectorSubcoreMesh(core_axis_name="c", subcore_axis_name="s")

    @functools.partial(
        pl.kernel, mesh=mesh,
        out_type=[jax.ShapeDtypeStruct((nc, n_nodes, d_feat), jnp.float32),
                  jax.ShapeDtypeStruct((nc * n_nodes,), jnp.float32)],
        scratch_types=[
            pltpu.VMEM((K, d_feat), jnp.float32),
            pltpu.VMEM((K, d_feat), jnp.float32),
            pltpu.VMEM((K, d_feat), jnp.float32),
            pltpu.VMEM((K, d_feat), jnp.float32),
            pltpu.VMEM((K,), jnp.float32),
            pltpu.VMEM((K,), jnp.float32),
            pltpu.VMEM((K,), jnp.int32),
            pltpu.VMEM((K,), jnp.int32),
            pltpu.VMEM((K,), jnp.int32),
            pltpu.VMEM((K,), jnp.int32),
            pltpu.VMEM((ZR, d_feat), jnp.float32),
            pltpu.VMEM((share,), jnp.float32),
            pltpu.VMEM_SHARED((n_nodes, d_feat), jnp.float32),
            pltpu.VMEM_SHARED((n_nodes,), jnp.float32),
            pltpu.SemaphoreType.DMA,
            pltpu.SemaphoreType.DMA,
            pltpu.SemaphoreType.DMA,
            pltpu.SemaphoreType.DMA,
        ],
    )
    def sc_scatter(x_hbm, w_hbm, d_hbm, snd_hbm, rcv_hbm, msg_out, dens_out,
          xrows0, xrows1, wrows0, wrows1, drows0, drows1,
          sidx0, sidx1, ridx0, ridx1, zbuf, zdbuf, msg_acc, dens_acc,
          sem0, sem1, isem0, isem1):
        cid = lax.axis_index("c")
        sid = lax.axis_index("s")
        wid = sid * nc + cid
        bufs = ((xrows0, wrows0, drows0, sidx0, ridx0, sem0, isem0),
                (xrows1, wrows1, drows1, sidx1, ridx1, sem1, isem1))
        zero16 = jnp.zeros((16,), jnp.float32)

        def zb(r, _):
            for g in range(d_feat // 16):
                zbuf[r, pl.ds(g * 16, 16)] = zero16
            return 0
        lax.fori_loop(0, ZR, zb, 0)

        def zd(r, _):
            zdbuf[pl.ds(r * 16, 16)] = zero16
            return 0
        lax.fori_loop(0, share // 16, zd, 0)

        row0 = sid * share
        for t in range(nz):
            pltpu.sync_copy(zbuf, msg_acc.at[pl.ds(row0 + t * ZR, ZR), :])
        pltpu.sync_copy(zdbuf, dens_acc.at[pl.ds(row0, share)])

        @pl.when(sid == ns - 1)
        def _zt():
            pltpu.sync_copy(zbuf.at[pl.ds(0, extra), :],
                            msg_acc.at[pl.ds(ns * share, extra), :])
            pltpu.sync_copy(zdbuf.at[pl.ds(0, extra)],
                            dens_acc.at[pl.ds(ns * share, extra)])

        plsc.subcore_barrier()

        base = wid * ept

        def fire_idx(i, b):
            sidx, ridx, isem = bufs[b][3], bufs[b][4], bufs[b][6]
            e0 = base + i * K
            pltpu.async_copy(snd_hbm.at[pl.ds(e0, K)], sidx, isem)
            pltpu.async_copy(rcv_hbm.at[pl.ds(e0, K)], ridx, isem)

        def drain_idx(i, b):
            sidx, ridx, isem = bufs[b][3], bufs[b][4], bufs[b][6]
            e0 = base + i * K
            pltpu.make_async_copy(snd_hbm.at[pl.ds(e0, K)], sidx, isem).wait()
            pltpu.make_async_copy(rcv_hbm.at[pl.ds(e0, K)], ridx, isem).wait()

        def fire_data(i, b):
            xrows, wrows, drows, sidx = bufs[b][0], bufs[b][1], bufs[b][2], bufs[b][3]
            sem = bufs[b][5]
            e0 = base + i * K
            pltpu.async_copy(x_hbm.at[sidx], xrows, sem)
            pltpu.async_copy(w_hbm.at[pl.ds(e0, K), :], wrows, sem)
            pltpu.async_copy(d_hbm.at[pl.ds(e0, K)], drows, sem)

        def process(i, b):
            xrows, wrows, drows, sidx, ridx, sem, isem = bufs[b]
            bo = 1 - b
            e0 = base + i * K
            pltpu.make_async_copy(x_hbm.at[sidx], xrows, sem).wait()
            pltpu.make_async_copy(w_hbm.at[pl.ds(e0, K), :], wrows, sem).wait()
            pltpu.make_async_copy(d_hbm.at[pl.ds(e0, K)], drows, sem).wait()

            def mulrow(r, _):
                for g in range(d_feat // 16):
                    c = g * 16
                    wrows[r, pl.ds(c, 16)] = (wrows[r, pl.ds(c, 16)]
                                              * xrows[r, pl.ds(c, 16)])
                return 0
            lax.fori_loop(0, K, mulrow, 0)
            pltpu.sync_copy(wrows, msg_acc.at[ridx], add=True)
            pltpu.sync_copy(drows, dens_acc.at[ridx], add=True)

            @pl.when(i + 2 < n_chunks)
            def _pfi():
                fire_idx(i + 2, b)

            @pl.when(i + 1 < n_chunks)
            def _pfd():
                drain_idx(i + 1, bo)
                fire_data(i + 1, bo)

        fire_idx(0, 0)
        fire_idx(1, 1)
        drain_idx(0, 0)
        fire_data(0, 0)

        @pl.loop(0, n_chunks, step=2)
        def _(i):
            process(i, 0)
            process(i + 1, 1)

        plsc.subcore_barrier()
        pltpu.sync_copy(msg_acc.at[pl.ds(row0, share), :],
                        msg_out.at[cid, pl.ds(row0, share), :])
        pltpu.sync_copy(dens_acc.at[pl.ds(row0, share)], zdbuf)
        pltpu.sync_copy(zdbuf, dens_out.at[pl.ds(cid * n_nodes + row0, share)])

        @pl.when(sid == ns - 1)
        def _pt():
            pltpu.sync_copy(msg_acc.at[pl.ds(ns * share, extra), :],
                            msg_out.at[cid, pl.ds(ns * share, extra), :])
            pltpu.sync_copy(dens_acc.at[pl.ds(ns * share, extra)],
                            zdbuf.at[pl.ds(0, extra)])
            pltpu.sync_copy(zdbuf.at[pl.ds(0, extra)],
                            dens_out.at[pl.ds(cid * n_nodes + ns * share, extra)])

    return sc_scatter


def kernel(node_attrs, node_feats, edge_attrs, edge_feats, edge_index,
           W_up, W_mlp1, W_mlp2, W_mlp3, W_mlp4, W_lin, W_skip, W_dens,
           W_demb):
    n_nodes, d_feat = node_feats.shape
    n_edges, d_edge = edge_feats.shape
    d_attr = node_attrs.shape[1]

    f32 = jnp.float32
    w_up = W_up / math.sqrt(d_feat)
    w1 = W_mlp1 / math.sqrt(d_edge)
    w2 = W_mlp2 / math.sqrt(64.0)
    w3 = W_mlp3 / math.sqrt(64.0)
    w4 = W_mlp4 / math.sqrt(64.0)
    wd = W_dens / math.sqrt(d_edge)
    wlin = W_lin / math.sqrt(d_feat)
    wskip = jnp.transpose(W_skip, (1, 0, 2)) / math.sqrt(d_feat * d_attr)
    half = 16
    freqs = jnp.exp(-math.log(100.0)
                    * jnp.arange(half, dtype=f32) / half).reshape(1, half)
    wsin = W_demb[:half]
    wcos = W_demb[half:]

    BN = 1000
    x = pl.pallas_call(
        _up_kernel,
        grid=(n_nodes // BN,),
        in_specs=[
            pl.BlockSpec((BN, d_feat), lambda i: (i, 0)),
            pl.BlockSpec((d_feat, d_feat), lambda i: (0, 0)),
        ],
        out_specs=pl.BlockSpec((BN, d_feat), lambda i: (i, 0)),
        out_shape=jax.ShapeDtypeStruct((n_nodes, d_feat), f32),
    )(node_feats, w_up)

    EB = 2560
    w_edges, dens_edges = pl.pallas_call(
        _edge_kernel,
        grid=(n_edges // EB,),
        in_specs=[
            pl.BlockSpec((EB, d_edge), lambda i: (i, 0)),
            pl.BlockSpec((EB, 1), lambda i: (i, 0)),
            pl.BlockSpec((d_edge, 64), lambda i: (0, 0)),
            pl.BlockSpec((64, 64), lambda i: (0, 0)),
            pl.BlockSpec((64, 64), lambda i: (0, 0)),
            pl.BlockSpec((64, d_feat), lambda i: (0, 0)),
            pl.BlockSpec((d_edge, 1), lambda i: (0, 0)),
        ],
        out_specs=[
            pl.BlockSpec((EB, d_feat), lambda i: (i, 0)),
            pl.BlockSpec((EB, 1), lambda i: (i, 0)),
        ],
        out_shape=[
            jax.ShapeDtypeStruct((n_edges, d_feat), f32),
            jax.ShapeDtypeStruct((n_edges, 1), f32),
        ],
    )(edge_feats, edge_attrs, w1, w2, w3, w4, wd)

    info = plsc.get_sparse_core_info()
    nc, ns = info.num_cores, info.num_subcores
    sender = edge_index[0]
    receiver = edge_index[1]
    sc = _make_sc_scatter(n_nodes, d_feat, n_edges, nc, ns)
    msg_p, dens_p = sc(x, w_edges, dens_edges.reshape(n_edges),
                       sender, receiver)
    dens_p = dens_p.reshape(nc, n_nodes, 1)

    out = pl.pallas_call(
        _tail_kernel,
        grid=(n_nodes // BN,),
        in_specs=[
            pl.BlockSpec((nc, BN, d_feat), lambda i: (0, i, 0)),
            pl.BlockSpec((nc, BN, 1), lambda i: (0, i, 0)),
            pl.BlockSpec((BN, d_attr), lambda i: (i, 0)),
            pl.BlockSpec((1, half), lambda i: (0, 0)),
            pl.BlockSpec((half, d_feat), lambda i: (0, 0)),
            pl.BlockSpec((half, d_feat), lambda i: (0, 0)),
            pl.BlockSpec((d_feat, d_feat), lambda i: (0, 0)),
            pl.BlockSpec((d_attr, d_feat, d_feat), lambda i: (0, 0, 0)),
        ],
        out_specs=pl.BlockSpec((BN, d_feat), lambda i: (i, 0)),
        out_shape=jax.ShapeDtypeStruct((n_nodes, d_feat), f32),
    )(msg_p, dens_p, node_attrs, freqs, wsin, wcos, wlin, wskip)

    return out[:, :, None]


# speedup vs baseline: 2.3393x; 1.7433x over previous
import functools
import math

import jax
import jax.numpy as jnp
from jax import lax
from jax.experimental import pallas as pl
from jax.experimental.pallas import tpu as pltpu
from jax.experimental.pallas import tpu_sc as plsc


def _silu(x):
    return x * jax.nn.sigmoid(x)


def _up_kernel(nf_ref, w_ref, o_ref):
    o_ref[...] = jnp.dot(nf_ref[...], w_ref[...])


def _edge_kernel(ef_ref, ea_ref, w1_ref, w2_ref, w3_ref, w4_ref, wd_ref,
                 w_ref, dp_ref):
    ef = ef_ref[...]
    h = _silu(jnp.dot(ef, w1_ref[...]))
    h = _silu(jnp.dot(h, w2_ref[...]))
    h = _silu(jnp.dot(h, w3_ref[...]))
    tw = jnp.dot(h, w4_ref[...])
    w_ref[...] = tw * ea_ref[...]
    dp_ref[...] = jnp.tanh(jnp.dot(ef, wd_ref[...]) ** 2)


def _tail_kernel(msgp_ref, densp_ref, attr_ref, fr_ref, wsin_ref, wcos_ref,
                 wlin_ref, ws_ref, o_ref):
    msg = msgp_ref[0] + msgp_ref[1]
    dens = densp_ref[0] + densp_ref[1]
    message = msg / (dens + 1.0)
    args = dens * fr_ref[...]
    message = (message
               + jnp.dot(jnp.sin(args), wsin_ref[...])
               + jnp.dot(jnp.cos(args), wcos_ref[...]))
    t = jnp.dot(message, wlin_ref[...])
    acc = jnp.dot(t, ws_ref[0]) * attr_ref[:, 0:1]
    for v in range(1, ws_ref.shape[0]):
        acc = acc + (jnp.dot(t, ws_ref[v])
                     * attr_ref[:, v:v + 1])
    o_ref[...] = acc


def _make_sc_scatter(n_nodes, d_feat, n_edges, nc, ns):
    nw = nc * ns
    ept = n_edges // nw
    K = 40
    n_chunks = ept // K
    share = (n_nodes // ns) // 8 * 8
    extra = n_nodes - share * ns
    ZR = 16
    nz = share // ZR

    mesh = plsc.VectorSubcoreMesh(core_axis_name="c", subcore_axis_name="s")

    @functools.partial(
        pl.kernel, mesh=mesh,
        out_type=[jax.ShapeDtypeStruct((nc, n_nodes, d_feat), jnp.float32),
                  jax.ShapeDtypeStruct((nc * n_nodes,), jnp.float32)],
        scratch_types=[
            pltpu.VMEM((K, d_feat), jnp.float32),
            pltpu.VMEM((K, d_feat), jnp.float32),
            pltpu.VMEM((K, d_feat), jnp.float32),
            pltpu.VMEM((K, d_feat), jnp.float32),
            pltpu.VMEM((K,), jnp.float32),
            pltpu.VMEM((K,), jnp.float32),
            pltpu.VMEM((K,), jnp.int32),
            pltpu.VMEM((K,), jnp.int32),
            pltpu.VMEM((K,), jnp.int32),
            pltpu.VMEM((K,), jnp.int32),
            pltpu.VMEM((ZR, d_feat), jnp.float32),
            pltpu.VMEM((share,), jnp.float32),
            pltpu.VMEM_SHARED((n_nodes, d_feat), jnp.float32),
            pltpu.VMEM_SHARED((n_nodes,), jnp.float32),
            pltpu.SemaphoreType.DMA,
            pltpu.SemaphoreType.DMA,
            pltpu.SemaphoreType.DMA,
            pltpu.SemaphoreType.DMA,
        ],
    )
    def sc_scatter(x_hbm, w_hbm, d_hbm, snd_hbm, rcv_hbm, msg_out, dens_out,
          xrows0, xrows1, wrows0, wrows1, drows0, drows1,
          sidx0, sidx1, ridx0, ridx1, zbuf, zdbuf, msg_acc, dens_acc,
          sem0, sem1, isem0, isem1):
        cid = lax.axis_index("c")
        sid = lax.axis_index("s")
        wid = sid * nc + cid
        bufs = ((xrows0, wrows0, drows0, sidx0, ridx0, sem0, isem0),
                (xrows1, wrows1, drows1, sidx1, ridx1, sem1, isem1))
        zero16 = jnp.zeros((16,), jnp.float32)

        def zb(r, _):
            for g in range(d_feat // 16):
                zbuf[r, pl.ds(g * 16, 16)] = zero16
            return 0
        lax.fori_loop(0, ZR, zb, 0)

        def zd(r, _):
            zdbuf[pl.ds(r * 16, 16)] = zero16
            return 0
        lax.fori_loop(0, share // 16, zd, 0)

        row0 = sid * share
        for t in range(nz):
            pltpu.sync_copy(zbuf, msg_acc.at[pl.ds(row0 + t * ZR, ZR), :])
        pltpu.sync_copy(zdbuf, dens_acc.at[pl.ds(row0, share)])

        @pl.when(sid == ns - 1)
        def _zt():
            pltpu.sync_copy(zbuf.at[pl.ds(0, extra), :],
                            msg_acc.at[pl.ds(ns * share, extra), :])
            pltpu.sync_copy(zdbuf.at[pl.ds(0, extra)],
                            dens_acc.at[pl.ds(ns * share, extra)])

        plsc.subcore_barrier()

        base = wid * ept

        def fire_idx(i, b):
            sidx, ridx, isem = bufs[b][3], bufs[b][4], bufs[b][6]
            e0 = base + i * K
            pltpu.async_copy(snd_hbm.at[pl.ds(e0, K)], sidx, isem)
            pltpu.async_copy(rcv_hbm.at[pl.ds(e0, K)], ridx, isem)

        def drain_idx(i, b):
            sidx, ridx, isem = bufs[b][3], bufs[b][4], bufs[b][6]
            e0 = base + i * K
            pltpu.make_async_copy(snd_hbm.at[pl.ds(e0, K)], sidx, isem).wait()
            pltpu.make_async_copy(rcv_hbm.at[pl.ds(e0, K)], ridx, isem).wait()

        def fire_data(i, b):
            xrows, wrows, drows, sidx = bufs[b][0], bufs[b][1], bufs[b][2], bufs[b][3]
            sem = bufs[b][5]
            e0 = base + i * K
            pltpu.async_copy(x_hbm.at[sidx], xrows, sem)
            pltpu.async_copy(w_hbm.at[pl.ds(e0, K), :], wrows, sem)
            pltpu.async_copy(d_hbm.at[pl.ds(e0, K)], drows, sem)

        def process(i, b):
            xrows, wrows, drows, sidx, ridx, sem, isem = bufs[b]
            bo = 1 - b
            e0 = base + i * K
            pltpu.make_async_copy(x_hbm.at[sidx], xrows, sem).wait()
            pltpu.make_async_copy(w_hbm.at[pl.ds(e0, K), :], wrows, sem).wait()
            pltpu.make_async_copy(d_hbm.at[pl.ds(e0, K)], drows, sem).wait()

            def mulrow(r, _):
                for g in range(d_feat // 16):
                    c = g * 16
                    wrows[r, pl.ds(c, 16)] = (wrows[r, pl.ds(c, 16)]
                                              * xrows[r, pl.ds(c, 16)])
                return 0
            lax.fori_loop(0, K, mulrow, 0)
            pltpu.sync_copy(wrows, msg_acc.at[ridx], add=True)
            pltpu.sync_copy(drows, dens_acc.at[ridx], add=True)

            @pl.when(i + 2 < n_chunks)
            def _pfi():
                fire_idx(i + 2, b)

            @pl.when(i + 1 < n_chunks)
            def _pfd():
                drain_idx(i + 1, bo)
                fire_data(i + 1, bo)

        fire_idx(0, 0)
        fire_idx(1, 1)
        drain_idx(0, 0)
        fire_data(0, 0)

        @pl.loop(0, n_chunks, step=2)
        def _(i):
            process(i, 0)
            process(i + 1, 1)

        plsc.subcore_barrier()
        pltpu.sync_copy(msg_acc.at[pl.ds(row0, share), :],
                        msg_out.at[cid, pl.ds(row0, share), :])
        pltpu.sync_copy(dens_acc.at[pl.ds(row0, share)], zdbuf)
        pltpu.sync_copy(zdbuf, dens_out.at[pl.ds(cid * n_nodes + row0, share)])

        @pl.when(sid == ns - 1)
        def _pt():
            pltpu.sync_copy(msg_acc.at[pl.ds(ns * share, extra), :],
                            msg_out.at[cid, pl.ds(ns * share, extra), :])
            pltpu.sync_copy(dens_acc.at[pl.ds(ns * share, extra)],
                            zdbuf.at[pl.ds(0, extra)])
            pltpu.sync_copy(zdbuf.at[pl.ds(0, extra)],
                            dens_out.at[pl.ds(cid * n_nodes + ns * share, extra)])

    return sc_scatter


def kernel(node_attrs, node_feats, edge_attrs, edge_feats, edge_index,
           W_up, W_mlp1, W_mlp2, W_mlp3, W_mlp4, W_lin, W_skip, W_dens,
           W_demb):
    n_nodes, d_feat = node_feats.shape
    n_edges, d_edge = edge_feats.shape
    d_attr = node_attrs.shape[1]

    f32 = jnp.float32
    w_up = W_up / math.sqrt(d_feat)
    w1 = W_mlp1 / math.sqrt(d_edge)
    w2 = W_mlp2 / math.sqrt(64.0)
    w3 = W_mlp3 / math.sqrt(64.0)
    w4 = W_mlp4 / math.sqrt(64.0)
    wd = W_dens / math.sqrt(d_edge)
    wlin = W_lin / math.sqrt(d_feat)
    wskip = jnp.transpose(W_skip, (1, 0, 2)) / math.sqrt(d_feat * d_attr)
    half = 16
    freqs = jnp.exp(-math.log(100.0)
                    * jnp.arange(half, dtype=f32) / half).reshape(1, half)
    wsin = W_demb[:half]
    wcos = W_demb[half:]

    BN = 1000
    x = pl.pallas_call(
        _up_kernel,
        grid=(n_nodes // BN,),
        in_specs=[
            pl.BlockSpec((BN, d_feat), lambda i: (i, 0)),
            pl.BlockSpec((d_feat, d_feat), lambda i: (0, 0)),
        ],
        out_specs=pl.BlockSpec((BN, d_feat), lambda i: (i, 0)),
        out_shape=jax.ShapeDtypeStruct((n_nodes, d_feat), f32),
    )(node_feats, w_up)

    EB = 2560
    w_edges, dens_edges = pl.pallas_call(
        _edge_kernel,
        grid=(n_edges // EB,),
        in_specs=[
            pl.BlockSpec((EB, d_edge), lambda i: (i, 0)),
            pl.BlockSpec((EB, 1), lambda i: (i, 0)),
            pl.BlockSpec((d_edge, 64), lambda i: (0, 0)),
            pl.BlockSpec((64, 64), lambda i: (0, 0)),
            pl.BlockSpec((64, 64), lambda i: (0, 0)),
            pl.BlockSpec((64, d_feat), lambda i: (0, 0)),
            pl.BlockSpec((d_edge, 1), lambda i: (0, 0)),
        ],
        out_specs=[
            pl.BlockSpec((EB, d_feat), lambda i: (i, 0)),
            pl.BlockSpec((EB, 1), lambda i: (i, 0)),
        ],
        out_shape=[
            jax.ShapeDtypeStruct((n_edges, d_feat), f32),
            jax.ShapeDtypeStruct((n_edges, 1), f32),
        ],
    )(edge_feats, edge_attrs, w1, w2, w3, w4, wd)

    info = plsc.get_sparse_core_info()
    nc, ns = info.num_cores, info.num_subcores
    sender = edge_index[0]
    receiver = edge_index[1]
    sc = _make_sc_scatter(n_nodes, d_feat, n_edges, nc, ns)
    msg_p, dens_p = sc(x, w_edges, dens_edges.reshape(n_edges),
                       sender, receiver)
    dens_p = dens_p.reshape(nc, n_nodes, 1)

    out = pl.pallas_call(
        _tail_kernel,
        grid=(n_nodes // BN,),
        in_specs=[
            pl.BlockSpec((nc, BN, d_feat), lambda i: (0, i, 0)),
            pl.BlockSpec((nc, BN, 1), lambda i: (0, i, 0)),
            pl.BlockSpec((BN, d_attr), lambda i: (i, 0)),
            pl.BlockSpec((1, half), lambda i: (0, 0)),
            pl.BlockSpec((half, d_feat), lambda i: (0, 0)),
            pl.BlockSpec((half, d_feat), lambda i: (0, 0)),
            pl.BlockSpec((d_feat, d_feat), lambda i: (0, 0)),
            pl.BlockSpec((d_attr, d_feat, d_feat), lambda i: (0, 0, 0)),
        ],
        out_specs=pl.BlockSpec((BN, d_feat), lambda i: (i, 0)),
        out_shape=jax.ShapeDtypeStruct((n_nodes, d_feat), f32),
    )(msg_p, dens_p, node_attrs, freqs, wsin, wcos, wlin, wskip)

    return out[:, :, None]


# speedup vs baseline: 3.1822x; 1.3603x over previous
import functools
import math

import jax
import jax.numpy as jnp
from jax import lax
from jax.experimental import pallas as pl
from jax.experimental.pallas import tpu as pltpu
from jax.experimental.pallas import tpu_sc as plsc


def _silu(x):
    return x * jax.nn.sigmoid(x)


def _up_kernel(nf_ref, w_ref, o_ref, *, scale):
    o_ref[...] = jnp.dot(nf_ref[...], w_ref[...]) * scale


def _edge_kernel(eft_ref, eat_ref, w1_ref, w2_ref, w3_ref, w4_ref, wdt_ref,
                 w_ref, dpt_ref, *, s_in, s_mid):
    eft = eft_ref[...]
    ef = jnp.transpose(eft)
    h = _silu(jnp.dot(ef, w1_ref[...]) * s_in)
    h = _silu(jnp.dot(h, w2_ref[...]) * s_mid)
    h = _silu(jnp.dot(h, w3_ref[...]) * s_mid)
    tw = jnp.dot(h, w4_ref[...]) * s_mid
    w_ref[...] = tw * jnp.transpose(eat_ref[...])
    dpt_ref[...] = jnp.tanh((jnp.dot(wdt_ref[...], eft) * s_in) ** 2)


def _tail_kernel(msgp_ref, densp_ref, attr_ref, fr_ref, wsin_ref, wcos_ref,
                 wlin_ref, ws_ref, o_ref, *, s_lin, s_skip, n_attr):
    msg = msgp_ref[0] + msgp_ref[1]
    dens = densp_ref[0] + densp_ref[1]
    message = msg / (dens + 1.0)
    args = dens * fr_ref[...]
    message = (message
               + jnp.dot(jnp.sin(args), wsin_ref[...])
               + jnp.dot(jnp.cos(args), wcos_ref[...]))
    t = jnp.dot(message, wlin_ref[...]) * s_lin
    p = jnp.concatenate([t * attr_ref[:, v:v + 1] for v in range(n_attr)],
                        axis=1)
    o_ref[...] = jnp.dot(p, ws_ref[...]) * s_skip


def _make_sc_scatter(n_nodes, d_feat, n_edges, nc, ns):
    nw = nc * ns
    ept = n_edges // nw
    K = 40
    n_chunks = ept // K
    share = (n_nodes // ns) // 8 * 8
    extra = n_nodes - share * ns
    ZR = 16
    nz = share // ZR

    mesh = plsc.VectorSubcoreMesh(core_axis_name="c", subcore_axis_name="s")

    @functools.partial(
        pl.kernel, mesh=mesh,
        out_type=[jax.ShapeDtypeStruct((nc, n_nodes, d_feat), jnp.float32),
                  jax.ShapeDtypeStruct((nc * n_nodes,), jnp.float32)],
        scratch_types=[
            pltpu.VMEM((K, d_feat), jnp.float32),
            pltpu.VMEM((K, d_feat), jnp.float32),
            pltpu.VMEM((K, d_feat), jnp.float32),
            pltpu.VMEM((K, d_feat), jnp.float32),
            pltpu.VMEM((K,), jnp.float32),
            pltpu.VMEM((K,), jnp.float32),
            pltpu.VMEM((K,), jnp.int32),
            pltpu.VMEM((K,), jnp.int32),
            pltpu.VMEM((K,), jnp.int32),
            pltpu.VMEM((K,), jnp.int32),
            pltpu.VMEM((ZR, d_feat), jnp.float32),
            pltpu.VMEM((share,), jnp.float32),
            pltpu.VMEM_SHARED((n_nodes, d_feat), jnp.float32),
            pltpu.VMEM_SHARED((n_nodes,), jnp.float32),
            pltpu.SemaphoreType.DMA,
            pltpu.SemaphoreType.DMA,
            pltpu.SemaphoreType.DMA,
            pltpu.SemaphoreType.DMA,
        ],
    )
    def sc_scatter(x_hbm, w_hbm, d_hbm, snd_hbm, rcv_hbm, msg_out, dens_out,
          xrows0, xrows1, wrows0, wrows1, drows0, drows1,
          sidx0, sidx1, ridx0, ridx1, zbuf, zdbuf, msg_acc, dens_acc,
          sem0, sem1, isem0, isem1):
        cid = lax.axis_index("c")
        sid = lax.axis_index("s")
        wid = sid * nc + cid
        bufs = ((xrows0, wrows0, drows0, sidx0, ridx0, sem0, isem0),
                (xrows1, wrows1, drows1, sidx1, ridx1, sem1, isem1))
        zero16 = jnp.zeros((16,), jnp.float32)

        def zb(r, _):
            for g in range(d_feat // 16):
                zbuf[r, pl.ds(g * 16, 16)] = zero16
            return 0
        lax.fori_loop(0, ZR, zb, 0)

        def zd(r, _):
            zdbuf[pl.ds(r * 16, 16)] = zero16
            return 0
        lax.fori_loop(0, share // 16, zd, 0)

        row0 = sid * share
        for t in range(nz):
            pltpu.sync_copy(zbuf, msg_acc.at[pl.ds(row0 + t * ZR, ZR), :])
        pltpu.sync_copy(zdbuf, dens_acc.at[pl.ds(row0, share)])

        @pl.when(sid == ns - 1)
        def _zt():
            pltpu.sync_copy(zbuf.at[pl.ds(0, extra), :],
                            msg_acc.at[pl.ds(ns * share, extra), :])
            pltpu.sync_copy(zdbuf.at[pl.ds(0, extra)],
                            dens_acc.at[pl.ds(ns * share, extra)])

        plsc.subcore_barrier()

        base = wid * ept

        def fire_idx(i, b):
            sidx, ridx, isem = bufs[b][3], bufs[b][4], bufs[b][6]
            e0 = base + i * K
            pltpu.async_copy(snd_hbm.at[pl.ds(e0, K)], sidx, isem)
            pltpu.async_copy(rcv_hbm.at[pl.ds(e0, K)], ridx, isem)

        def drain_idx(i, b):
            sidx, ridx, isem = bufs[b][3], bufs[b][4], bufs[b][6]
            e0 = base + i * K
            pltpu.make_async_copy(snd_hbm.at[pl.ds(e0, K)], sidx, isem).wait()
            pltpu.make_async_copy(rcv_hbm.at[pl.ds(e0, K)], ridx, isem).wait()

        def fire_data(i, b):
            xrows, wrows, drows, sidx = bufs[b][0], bufs[b][1], bufs[b][2], bufs[b][3]
            sem = bufs[b][5]
            e0 = base + i * K
            pltpu.async_copy(x_hbm.at[sidx], xrows, sem)
            pltpu.async_copy(w_hbm.at[pl.ds(e0, K), :], wrows, sem)
            pltpu.async_copy(d_hbm.at[pl.ds(e0, K)], drows, sem)

        def process(i, b):
            xrows, wrows, drows, sidx, ridx, sem, isem = bufs[b]
            bo = 1 - b
            e0 = base + i * K
            pltpu.make_async_copy(x_hbm.at[sidx], xrows, sem).wait()
            pltpu.make_async_copy(w_hbm.at[pl.ds(e0, K), :], wrows, sem).wait()
            pltpu.make_async_copy(d_hbm.at[pl.ds(e0, K)], drows, sem).wait()

            def mulrow(r, _):
                for g in range(d_feat // 16):
                    c = g * 16
                    wrows[r, pl.ds(c, 16)] = (wrows[r, pl.ds(c, 16)]
                                              * xrows[r, pl.ds(c, 16)])
                return 0
            lax.fori_loop(0, K, mulrow, 0)
            pltpu.sync_copy(wrows, msg_acc.at[ridx], add=True)
            pltpu.sync_copy(drows, dens_acc.at[ridx], add=True)

            @pl.when(i + 2 < n_chunks)
            def _pfi():
                fire_idx(i + 2, b)

            @pl.when(i + 1 < n_chunks)
            def _pfd():
                drain_idx(i + 1, bo)
                fire_data(i + 1, bo)

        fire_idx(0, 0)
        fire_idx(1, 1)
        drain_idx(0, 0)
        fire_data(0, 0)

        @pl.loop(0, n_chunks, step=2)
        def _(i):
            process(i, 0)
            process(i + 1, 1)

        plsc.subcore_barrier()
        pltpu.sync_copy(msg_acc.at[pl.ds(row0, share), :],
                        msg_out.at[cid, pl.ds(row0, share), :])
        pltpu.sync_copy(dens_acc.at[pl.ds(row0, share)], zdbuf)
        pltpu.sync_copy(zdbuf, dens_out.at[pl.ds(cid * n_nodes + row0, share)])

        @pl.when(sid == ns - 1)
        def _pt():
            pltpu.sync_copy(msg_acc.at[pl.ds(ns * share, extra), :],
                            msg_out.at[cid, pl.ds(ns * share, extra), :])
            pltpu.sync_copy(dens_acc.at[pl.ds(ns * share, extra)],
                            zdbuf.at[pl.ds(0, extra)])
            pltpu.sync_copy(zdbuf.at[pl.ds(0, extra)],
                            dens_out.at[pl.ds(cid * n_nodes + ns * share, extra)])

    return sc_scatter


def kernel(node_attrs, node_feats, edge_attrs, edge_feats, edge_index,
           W_up, W_mlp1, W_mlp2, W_mlp3, W_mlp4, W_lin, W_skip, W_dens,
           W_demb):
    n_nodes, d_feat = node_feats.shape
    n_edges, d_edge = edge_feats.shape
    d_attr = node_attrs.shape[1]

    f32 = jnp.float32
    s_feat = 1.0 / math.sqrt(d_feat)
    s_edge = 1.0 / math.sqrt(d_edge)
    s_mid = 1.0 / math.sqrt(64.0)
    s_skip = 1.0 / math.sqrt(d_feat * d_attr)
    wskip2 = jnp.transpose(W_skip, (1, 0, 2)).reshape(d_attr * d_feat, d_feat)
    half = 16
    freqs = jnp.exp(-math.log(100.0)
                    * jnp.arange(half, dtype=f32) / half).reshape(1, half)
    wsin = W_demb[:half]
    wcos = W_demb[half:]

    BN = 1000
    x = pl.pallas_call(
        functools.partial(_up_kernel, scale=s_feat),
        grid=(n_nodes // BN,),
        in_specs=[
            pl.BlockSpec((BN, d_feat), lambda i: (i, 0)),
            pl.BlockSpec((d_feat, d_feat), lambda i: (0, 0)),
        ],
        out_specs=pl.BlockSpec((BN, d_feat), lambda i: (i, 0)),
        out_shape=jax.ShapeDtypeStruct((n_nodes, d_feat), f32),
    )(node_feats, W_up)

    EB = 2560
    w_edges, dens_t = pl.pallas_call(
        functools.partial(_edge_kernel, s_in=s_edge, s_mid=s_mid),
        grid=(n_edges // EB,),
        in_specs=[
            pl.BlockSpec((d_edge, EB), lambda i: (0, i)),
            pl.BlockSpec((1, EB), lambda i: (0, i)),
            pl.BlockSpec((d_edge, 64), lambda i: (0, 0)),
            pl.BlockSpec((64, 64), lambda i: (0, 0)),
            pl.BlockSpec((64, 64), lambda i: (0, 0)),
            pl.BlockSpec((64, d_feat), lambda i: (0, 0)),
            pl.BlockSpec((1, d_edge), lambda i: (0, 0)),
        ],
        out_specs=[
            pl.BlockSpec((EB, d_feat), lambda i: (i, 0)),
            pl.BlockSpec((1, EB), lambda i: (0, i)),
        ],
        out_shape=[
            jax.ShapeDtypeStruct((n_edges, d_feat), f32),
            jax.ShapeDtypeStruct((1, n_edges), f32),
        ],
    )(edge_feats.T, edge_attrs.T, W_mlp1, W_mlp2, W_mlp3, W_mlp4, W_dens.T)

    info = plsc.get_sparse_core_info()
    nc, ns = info.num_cores, info.num_subcores
    sender = edge_index[0]
    receiver = edge_index[1]
    sc = _make_sc_scatter(n_nodes, d_feat, n_edges, nc, ns)
    msg_p, dens_p = sc(x, w_edges, dens_t.reshape(n_edges),
                       sender, receiver)
    dens_p = dens_p.reshape(nc, n_nodes, 1)

    out = pl.pallas_call(
        functools.partial(_tail_kernel, s_lin=s_feat, s_skip=s_skip,
                          n_attr=d_attr),
        grid=(n_nodes // BN,),
        in_specs=[
            pl.BlockSpec((nc, BN, d_feat), lambda i: (0, i, 0)),
            pl.BlockSpec((nc, BN, 1), lambda i: (0, i, 0)),
            pl.BlockSpec((BN, d_attr), lambda i: (i, 0)),
            pl.BlockSpec((1, half), lambda i: (0, 0)),
            pl.BlockSpec((half, d_feat), lambda i: (0, 0)),
            pl.BlockSpec((half, d_feat), lambda i: (0, 0)),
            pl.BlockSpec((d_feat, d_feat), lambda i: (0, 0)),
            pl.BlockSpec((d_attr * d_feat, d_feat), lambda i: (0, 0)),
        ],
        out_specs=pl.BlockSpec((BN, d_feat), lambda i: (i, 0)),
        out_shape=jax.ShapeDtypeStruct((n_nodes, d_feat), f32),
    )(msg_p, dens_p, node_attrs, freqs, wsin, wcos, W_lin, wskip2)

    return out[:, :, None]


# speedup vs baseline: 3.6612x; 1.1505x over previous
import functools
import math

import jax
import jax.numpy as jnp
from jax import lax
from jax.experimental import pallas as pl
from jax.experimental.pallas import tpu as pltpu
from jax.experimental.pallas import tpu_sc as plsc


def _silu(x):
    return x * jax.nn.sigmoid(x)


def _up_kernel(nf_ref, w_ref, o_ref, *, scale):
    o_ref[...] = jnp.dot(nf_ref[...], w_ref[...]) * scale


def _edge_kernel(eft_ref, eat_ref, w1_ref, w2_ref, w3_ref, w4_ref, wdt_ref,
                 w_ref, dpt_ref, *, s_in, s_mid):
    eft = eft_ref[...]
    ef = jnp.transpose(eft)
    h = _silu(jnp.dot(ef, w1_ref[...]) * s_in)
    h = _silu(jnp.dot(h, w2_ref[...]) * s_mid)
    h = _silu(jnp.dot(h, w3_ref[...]) * s_mid)
    tw = jnp.dot(h, w4_ref[...]) * s_mid
    w_ref[...] = tw * jnp.transpose(eat_ref[...])
    dpt_ref[...] = jnp.tanh((jnp.dot(wdt_ref[...], eft) * s_in) ** 2)


def _tail_kernel(msgp_ref, densp_ref, attr_ref, fr_ref, wsin_ref, wcos_ref,
                 wlin_ref, ws_ref, o_ref, *, s_lin, s_skip, n_attr):
    msg = msgp_ref[0] + msgp_ref[1]
    dens = densp_ref[0] + densp_ref[1]
    message = msg / (dens + 1.0)
    args = dens * fr_ref[...]
    message = (message
               + jnp.dot(jnp.sin(args), wsin_ref[...])
               + jnp.dot(jnp.cos(args), wcos_ref[...]))
    t = jnp.dot(message, wlin_ref[...]) * s_lin
    p = jnp.concatenate([t * attr_ref[:, v:v + 1] for v in range(n_attr)],
                        axis=1)
    o_ref[...] = jnp.dot(p, ws_ref[...]) * s_skip


def _make_sc_scatter(n_nodes, d_feat, n_edges, nc, ns):
    nw = nc * ns
    ept = n_edges // nw
    K = 80
    n_chunks = ept // K
    share = (n_nodes // ns) // 8 * 8
    extra = n_nodes - share * ns
    ZR = 16
    nz = share // ZR

    mesh = plsc.VectorSubcoreMesh(core_axis_name="c", subcore_axis_name="s")

    @functools.partial(
        pl.kernel, mesh=mesh,
        out_type=[jax.ShapeDtypeStruct((nc, n_nodes, d_feat), jnp.float32),
                  jax.ShapeDtypeStruct((nc * n_nodes,), jnp.float32)],
        scratch_types=[
            pltpu.VMEM((K, d_feat), jnp.float32),
            pltpu.VMEM((K, d_feat), jnp.float32),
            pltpu.VMEM((K, d_feat), jnp.float32),
            pltpu.VMEM((K, d_feat), jnp.float32),
            pltpu.VMEM((K,), jnp.float32),
            pltpu.VMEM((K,), jnp.float32),
            pltpu.VMEM((K,), jnp.int32),
            pltpu.VMEM((K,), jnp.int32),
            pltpu.VMEM((K,), jnp.int32),
            pltpu.VMEM((K,), jnp.int32),
            pltpu.VMEM((ZR, d_feat), jnp.float32),
            pltpu.VMEM((share,), jnp.float32),
            pltpu.VMEM_SHARED((n_nodes, d_feat), jnp.float32),
            pltpu.VMEM_SHARED((n_nodes,), jnp.float32),
            pltpu.SemaphoreType.DMA,
            pltpu.SemaphoreType.DMA,
            pltpu.SemaphoreType.DMA,
            pltpu.SemaphoreType.DMA,
        ],
    )
    def sc_scatter(x_hbm, w_hbm, d_hbm, snd_hbm, rcv_hbm, msg_out, dens_out,
          xrows0, xrows1, wrows0, wrows1, drows0, drows1,
          sidx0, sidx1, ridx0, ridx1, zbuf, zdbuf, msg_acc, dens_acc,
          sem0, sem1, isem0, isem1):
        cid = lax.axis_index("c")
        sid = lax.axis_index("s")
        wid = sid * nc + cid
        bufs = ((xrows0, wrows0, drows0, sidx0, ridx0, sem0, isem0),
                (xrows1, wrows1, drows1, sidx1, ridx1, sem1, isem1))
        zero16 = jnp.zeros((16,), jnp.float32)

        def zb(r, _):
            for g in range(d_feat // 16):
                zbuf[r, pl.ds(g * 16, 16)] = zero16
            return 0
        lax.fori_loop(0, ZR, zb, 0)

        def zd(r, _):
            zdbuf[pl.ds(r * 16, 16)] = zero16
            return 0
        lax.fori_loop(0, share // 16, zd, 0)

        row0 = sid * share
        for t in range(nz):
            pltpu.sync_copy(zbuf, msg_acc.at[pl.ds(row0 + t * ZR, ZR), :])
        pltpu.sync_copy(zdbuf, dens_acc.at[pl.ds(row0, share)])

        @pl.when(sid == ns - 1)
        def _zt():
            pltpu.sync_copy(zbuf.at[pl.ds(0, extra), :],
                            msg_acc.at[pl.ds(ns * share, extra), :])
            pltpu.sync_copy(zdbuf.at[pl.ds(0, extra)],
                            dens_acc.at[pl.ds(ns * share, extra)])

        plsc.subcore_barrier()

        base = wid * ept

        def fire_idx(i, b):
            sidx, ridx, isem = bufs[b][3], bufs[b][4], bufs[b][6]
            e0 = base + i * K
            pltpu.async_copy(snd_hbm.at[pl.ds(e0, K)], sidx, isem)
            pltpu.async_copy(rcv_hbm.at[pl.ds(e0, K)], ridx, isem)

        def drain_idx(i, b):
            sidx, ridx, isem = bufs[b][3], bufs[b][4], bufs[b][6]
            e0 = base + i * K
            pltpu.make_async_copy(snd_hbm.at[pl.ds(e0, K)], sidx, isem).wait()
            pltpu.make_async_copy(rcv_hbm.at[pl.ds(e0, K)], ridx, isem).wait()

        def fire_data(i, b):
            xrows, wrows, drows, sidx = bufs[b][0], bufs[b][1], bufs[b][2], bufs[b][3]
            sem = bufs[b][5]
            e0 = base + i * K
            pltpu.async_copy(x_hbm.at[sidx], xrows, sem)
            pltpu.async_copy(w_hbm.at[pl.ds(e0, K), :], wrows, sem)
            pltpu.async_copy(d_hbm.at[pl.ds(e0, K)], drows, sem)

        def process(i, b):
            xrows, wrows, drows, sidx, ridx, sem, isem = bufs[b]
            bo = 1 - b
            e0 = base + i * K
            pltpu.make_async_copy(x_hbm.at[sidx], xrows, sem).wait()
            pltpu.make_async_copy(w_hbm.at[pl.ds(e0, K), :], wrows, sem).wait()
            pltpu.make_async_copy(d_hbm.at[pl.ds(e0, K)], drows, sem).wait()

            def mulrow(r, _):
                for g in range(d_feat // 16):
                    c = g * 16
                    wrows[r, pl.ds(c, 16)] = (wrows[r, pl.ds(c, 16)]
                                              * xrows[r, pl.ds(c, 16)])
                return 0
            lax.fori_loop(0, K, mulrow, 0)
            pltpu.sync_copy(wrows, msg_acc.at[ridx], add=True)
            pltpu.sync_copy(drows, dens_acc.at[ridx], add=True)

            @pl.when(i + 2 < n_chunks)
            def _pfi():
                fire_idx(i + 2, b)

            @pl.when(i + 1 < n_chunks)
            def _pfd():
                drain_idx(i + 1, bo)
                fire_data(i + 1, bo)

        fire_idx(0, 0)
        fire_idx(1, 1)
        drain_idx(0, 0)
        fire_data(0, 0)

        n_even = n_chunks // 2 * 2

        @pl.loop(0, n_even, step=2)
        def _(i):
            process(i, 0)
            process(i + 1, 1)

        if n_chunks % 2:
            process(n_chunks - 1, 0)

        plsc.subcore_barrier()
        pltpu.sync_copy(msg_acc.at[pl.ds(row0, share), :],
                        msg_out.at[cid, pl.ds(row0, share), :])
        pltpu.sync_copy(dens_acc.at[pl.ds(row0, share)], zdbuf)
        pltpu.sync_copy(zdbuf, dens_out.at[pl.ds(cid * n_nodes + row0, share)])

        @pl.when(sid == ns - 1)
        def _pt():
            pltpu.sync_copy(msg_acc.at[pl.ds(ns * share, extra), :],
                            msg_out.at[cid, pl.ds(ns * share, extra), :])
            pltpu.sync_copy(dens_acc.at[pl.ds(ns * share, extra)],
                            zdbuf.at[pl.ds(0, extra)])
            pltpu.sync_copy(zdbuf.at[pl.ds(0, extra)],
                            dens_out.at[pl.ds(cid * n_nodes + ns * share, extra)])

    return sc_scatter


def kernel(node_attrs, node_feats, edge_attrs, edge_feats, edge_index,
           W_up, W_mlp1, W_mlp2, W_mlp3, W_mlp4, W_lin, W_skip, W_dens,
           W_demb):
    n_nodes, d_feat = node_feats.shape
    n_edges, d_edge = edge_feats.shape
    d_attr = node_attrs.shape[1]

    f32 = jnp.float32
    s_feat = 1.0 / math.sqrt(d_feat)
    s_edge = 1.0 / math.sqrt(d_edge)
    s_mid = 1.0 / math.sqrt(64.0)
    s_skip = 1.0 / math.sqrt(d_feat * d_attr)
    wskip2 = jnp.transpose(W_skip, (1, 0, 2)).reshape(d_attr * d_feat, d_feat)
    half = 16
    freqs = jnp.exp(-math.log(100.0)
                    * jnp.arange(half, dtype=f32) / half).reshape(1, half)
    wsin = W_demb[:half]
    wcos = W_demb[half:]

    BN = 1000
    x = pl.pallas_call(
        functools.partial(_up_kernel, scale=s_feat),
        grid=(n_nodes // BN,),
        in_specs=[
            pl.BlockSpec((BN, d_feat), lambda i: (i, 0)),
            pl.BlockSpec((d_feat, d_feat), lambda i: (0, 0)),
        ],
        out_specs=pl.BlockSpec((BN, d_feat), lambda i: (i, 0)),
        out_shape=jax.ShapeDtypeStruct((n_nodes, d_feat), f32),
    )(node_feats, W_up)

    EB = 2560
    w_edges, dens_t = pl.pallas_call(
        functools.partial(_edge_kernel, s_in=s_edge, s_mid=s_mid),
        grid=(n_edges // EB,),
        in_specs=[
            pl.BlockSpec((d_edge, EB), lambda i: (0, i)),
            pl.BlockSpec((1, EB), lambda i: (0, i)),
            pl.BlockSpec((d_edge, 64), lambda i: (0, 0)),
            pl.BlockSpec((64, 64), lambda i: (0, 0)),
            pl.BlockSpec((64, 64), lambda i: (0, 0)),
            pl.BlockSpec((64, d_feat), lambda i: (0, 0)),
            pl.BlockSpec((1, d_edge), lambda i: (0, 0)),
        ],
        out_specs=[
            pl.BlockSpec((EB, d_feat), lambda i: (i, 0)),
            pl.BlockSpec((1, EB), lambda i: (0, i)),
        ],
        out_shape=[
            jax.ShapeDtypeStruct((n_edges, d_feat), f32),
            jax.ShapeDtypeStruct((1, n_edges), f32),
        ],
    )(edge_feats.T, edge_attrs.T, W_mlp1, W_mlp2, W_mlp3, W_mlp4, W_dens.T)

    info = plsc.get_sparse_core_info()
    nc, ns = info.num_cores, info.num_subcores
    sender = edge_index[0]
    receiver = edge_index[1]
    sc = _make_sc_scatter(n_nodes, d_feat, n_edges, nc, ns)
    msg_p, dens_p = sc(x, w_edges, dens_t.reshape(n_edges),
                       sender, receiver)
    dens_p = dens_p.reshape(nc, n_nodes, 1)

    out = pl.pallas_call(
        functools.partial(_tail_kernel, s_lin=s_feat, s_skip=s_skip,
                          n_attr=d_attr),
        grid=(n_nodes // BN,),
        in_specs=[
            pl.BlockSpec((nc, BN, d_feat), lambda i: (0, i, 0)),
            pl.BlockSpec((nc, BN, 1), lambda i: (0, i, 0)),
            pl.BlockSpec((BN, d_attr), lambda i: (i, 0)),
            pl.BlockSpec((1, half), lambda i: (0, 0)),
            pl.BlockSpec((half, d_feat), lambda i: (0, 0)),
            pl.BlockSpec((half, d_feat), lambda i: (0, 0)),
            pl.BlockSpec((d_feat, d_feat), lambda i: (0, 0)),
            pl.BlockSpec((d_attr * d_feat, d_feat), lambda i: (0, 0)),
        ],
        out_specs=pl.BlockSpec((BN, d_feat), lambda i: (i, 0)),
        out_shape=jax.ShapeDtypeStruct((n_nodes, d_feat), f32),
    )(msg_p, dens_p, node_attrs, freqs, wsin, wcos, W_lin, wskip2)

    return out[:, :, None]
